# baseline (speedup 1.0000x reference)
import functools

import jax
import jax.numpy as jnp
from jax import lax
from jax.experimental import pallas as pl
from jax.experimental.pallas import tpu as pltpu
from jax.experimental.pallas import tpu_sc as plsc

NC = 2
NS = 16
NW = NC * NS
CH = 512
L = 16


V_BLK = 512


@functools.partial(jax.jit, static_argnums=(1, 2))
def _table_transpose(tt, v, d):
    n_blk = -(-v // V_BLK)
    nb_w = -(-n_blk // NW)
    nb_w += nb_w % 2
    n_pair = nb_w // 2
    mesh = plsc.VectorSubcoreMesh(
        core_axis_name="c", subcore_axis_name="s",
        num_cores=NC, num_subcores=NS)

    @functools.partial(
        pl.kernel,
        out_type=jax.ShapeDtypeStruct((v, d), jnp.float32),
        mesh=mesh,
        scratch_types=[
            [pltpu.VMEM((d, V_BLK), jnp.float32) for _ in range(2)],
            [pltpu.VMEM((V_BLK, d), jnp.float32) for _ in range(2)],
            [pltpu.SemaphoreType.DMA for _ in range(2)],
            [pltpu.SemaphoreType.DMA for _ in range(2)],
        ],
        compiler_params=pltpu.CompilerParams(
            use_tc_tiling_on_sc=False, needs_layout_passes=False),
    )
    def k(tt_hbm, lin_hbm, tin, tout, sem_r, sem_w):
        wid = lax.axis_index("s") * NC + lax.axis_index("c")
        g0 = wid * nb_w
        iota = lax.iota(jnp.int32, L)
        e_pat = [(iota + c) & (L - 1) for c in range(L)]

        def win(blk_i):
            return jnp.minimum(blk_i * V_BLK, v - V_BLK)

        def reads(blk_i, bb):
            start = win(blk_i)
            for dd in range(d):
                pltpu.async_copy(
                    tt_hbm.at[dd, pl.ds(start, V_BLK)], tin[bb].at[dd],
                    sem_r[bb])

        def wait_reads(bb):
            for dd in range(d):
                pltpu.make_async_copy(
                    tt_hbm.at[dd, pl.ds(0, V_BLK)], tin[bb].at[dd],
                    sem_r[bb]).wait()

        def transpose_blk(bb):
            src = tin[bb]
            dst = tout[bb]

            @plsc.parallel_loop(0, V_BLK // L, unroll=2)
            def body(vb):
                colbase = 16 * vb + iota
                for dd0 in (0, d // 2):
                    for c in range(L):
                        rc = e_pat[c] + dd0 if dd0 else e_pat[c]
                        val = plsc.load_gather(src, [rc, colbase])
                        plsc.store_scatter(dst, [colbase, rc], val)

        def write(blk_i, bb):
            pltpu.async_copy(
                tout[bb], lin_hbm.at[pl.ds(win(blk_i), V_BLK)], sem_w[bb])

        def wait_write(bb):
            pltpu.make_async_copy(
                tout[bb], lin_hbm.at[pl.ds(0, V_BLK)], sem_w[bb]).wait()

        reads(g0, 0)
        reads(g0 + 1, 1)
        write(g0, 0)
        write(g0 + 1, 1)

        def pair(p, carry):
            a = g0 + 2 * p
            wait_reads(0)
            wait_write(0)
            transpose_blk(0)
            write(a, 0)
            reads(jnp.minimum(a + 2, g0 + nb_w - 1), 0)
            wait_reads(1)
            wait_write(1)
            transpose_blk(1)
            write(a + 1, 1)
            reads(jnp.minimum(a + 3, g0 + nb_w - 1), 1)
            return carry

        lax.fori_loop(0, n_pair, pair, 0)
        wait_reads(0)
        wait_reads(1)
        wait_write(0)
        wait_write(1)

    return k(tt)


@functools.partial(jax.jit, static_argnums=(2, 3, 4))
def _emb_lookup(ids, table, n, d, b_sz):
    n_per_w = n // NW
    n_ch = n_per_w // CH
    n_pair = n_ch // 2
    ch_per_s = b_sz // CH
    dt = d // 8
    blk = CH * d // dt
    s_stride = b_sz * d
    mesh = plsc.VectorSubcoreMesh(
        core_axis_name="c", subcore_axis_name="s",
        num_cores=NC, num_subcores=NS)

    @functools.partial(
        pl.kernel,
        out_type=(
            jax.ShapeDtypeStruct((n * d,), jnp.float32),
            jax.ShapeDtypeStruct((blk,), jnp.float32),
        ),
        mesh=mesh,
        scratch_types=[
            [pltpu.VMEM((CH,), jnp.int32) for _ in range(2)],
            [pltpu.VMEM((CH, d), jnp.float32) for _ in range(2)],
            [pltpu.VMEM((CH * d,), jnp.float32) for _ in range(2)],
            [pltpu.SemaphoreType.DMA for _ in range(2)],
            [pltpu.SemaphoreType.DMA for _ in range(2)],
            [pltpu.SemaphoreType.DMA for _ in range(2)],
        ],
        compiler_params=pltpu.CompilerParams(
            use_tc_tiling_on_sc=False, needs_layout_passes=False),
    )
    def k(ids_hbm, table_hbm, out_hbm, dummy_hbm, idx_v, rows_v, t5, sem_i,
          sem_g, sem_s):
        wid = lax.axis_index("s") * NC + lax.axis_index("c")
        g0 = wid * n_ch
        iota = lax.iota(jnp.int32, L)
        e_pat = [(iota + c) & (L - 1) for c in range(L)]
        d_pat = [((e >> 3) << 12) + ((e & 7) << 7) + iota for e in e_pat]

        def idx_load(chunk, bb):
            return pltpu.async_copy(
                ids_hbm.at[pl.ds((g0 + chunk) * CH, CH)], idx_v[bb],
                sem_i[bb])

        def gather(bb):
            return pltpu.async_copy(
                table_hbm.at[idx_v[bb]], rows_v[bb], sem_g[bb])

        def transpose_chunk(bb):
            rows = rows_v[bb]
            dst = t5[bb]

            @plsc.parallel_loop(0, CH // L, unroll=2)
            def body(tb):
                row_idx = 16 * tb + iota
                btl = tb >> 3
                ttm = (tb & 7) << 4
                base0 = (btl << 10) + ttm
                for dd0 in (0, d // 2):
                    base = base0 + (dd0 >> 3) * blk
                    for c in range(L):
                        col = e_pat[c] + dd0 if dd0 else e_pat[c]
                        v = plsc.load_gather(rows, [row_idx, col])
                        plsc.store_scatter(dst, [d_pat[c] + base], v)

        def stores(chunk, bb):
            g = g0 + chunk
            s_idx = g // ch_per_s
            c = g - s_idx * ch_per_s
            hs = []
            for di in range(dt):
                dpos = s_idx * s_stride + di * blk * ch_per_s + c * blk
                hs.append(pltpu.async_copy(
                    t5[bb].at[pl.ds(di * blk, blk)],
                    out_hbm.at[pl.ds(dpos, blk)], sem_s[bb]))
            return hs

        def wait_stores(bb):
            for di in range(dt):
                pltpu.make_async_copy(
                    t5[bb].at[pl.ds(di * blk, blk)], dummy_hbm,
                    sem_s[bb]).wait()

        for bb in range(2):
            for di in range(dt):
                pltpu.async_copy(
                    t5[bb].at[pl.ds(di * blk, blk)], dummy_hbm, sem_s[bb])
        idx_load(0, 0)
        idx_load(1, 1)
        pltpu.make_async_copy(
            ids_hbm.at[pl.ds(0, CH)], idx_v[0], sem_i[0]).wait()
        gather(0)

        def pair(p, carry):
            a = 2 * p
            pltpu.make_async_copy(
                table_hbm.at[idx_v[0]], rows_v[0], sem_g[0]).wait()
            idx_load(jnp.minimum(a + 2, n_ch - 1), 0)
            pltpu.make_async_copy(
                ids_hbm.at[pl.ds(0, CH)], idx_v[1], sem_i[1]).wait()
            gather(1)
            wait_stores(0)
            transpose_chunk(0)
            stores(a, 0)
            pltpu.make_async_copy(
                table_hbm.at[idx_v[1]], rows_v[1], sem_g[1]).wait()
            idx_load(jnp.minimum(a + 3, n_ch - 1), 1)
            pltpu.make_async_copy(
                ids_hbm.at[pl.ds(0, CH)], idx_v[0], sem_i[0]).wait()
            gather(0)
            wait_stores(1)
            transpose_chunk(1)
            stores(a + 1, 1)
            return carry

        lax.fori_loop(0, n_pair, pair, 0)

        pltpu.make_async_copy(
            table_hbm.at[idx_v[0]], rows_v[0], sem_g[0]).wait()
        pltpu.make_async_copy(
            ids_hbm.at[pl.ds(0, CH)], idx_v[1], sem_i[1]).wait()
        wait_stores(0)
        wait_stores(1)

    return k(ids, table)


def kernel(token_ids, table):
    b, s = token_ids.shape
    v, d = table.shape
    n = b * s
    ids = token_ids.T.reshape(n).astype(jnp.int32)
    lin = _table_transpose(table.T, v, d)
    flat, _ = _emb_lookup(ids, lin, n, d, b)
    out5 = flat.reshape(s, d // 8, b // 128, 8, 128)
    out3 = out5.transpose(0, 1, 3, 2, 4).reshape(s, d, b)
    return out3.transpose(2, 0, 1)


# speedup vs baseline: 4.5205x; 4.5205x over previous
import functools

import jax
import jax.numpy as jnp
from jax import lax
from jax.experimental import pallas as pl
from jax.experimental.pallas import tpu as pltpu
from jax.experimental.pallas import tpu_sc as plsc

NC = 2
NS = 16
NW = NC * NS
CH = 512
L = 16


@functools.partial(jax.jit, static_argnums=(2, 3, 4))
def _emb_lookup(ids, table, n, d, b_sz):
    n_per_w = n // NW
    n_ch = n_per_w // CH
    n_pair = n_ch // 2
    ch_per_s = b_sz // CH
    dt = d // 8
    blk = CH * d // dt
    s_stride = b_sz * d
    mesh = plsc.VectorSubcoreMesh(
        core_axis_name="c", subcore_axis_name="s",
        num_cores=NC, num_subcores=NS)

    @functools.partial(
        pl.kernel,
        out_type=(
            jax.ShapeDtypeStruct((n * d,), jnp.float32),
            jax.ShapeDtypeStruct((blk,), jnp.float32),
        ),
        mesh=mesh,
        scratch_types=[
            [pltpu.VMEM((CH,), jnp.int32) for _ in range(2)],
            [pltpu.VMEM((CH, d), jnp.float32) for _ in range(2)],
            [pltpu.VMEM((CH * d,), jnp.float32) for _ in range(2)],
            [pltpu.SemaphoreType.DMA for _ in range(2)],
            [pltpu.SemaphoreType.DMA for _ in range(2)],
            [pltpu.SemaphoreType.DMA for _ in range(2)],
        ],
        compiler_params=pltpu.CompilerParams(
            use_tc_tiling_on_sc=False, needs_layout_passes=False),
    )
    def k(ids_hbm, table_hbm, out_hbm, dummy_hbm, idx_v, rows_v, t5, sem_i,
          sem_g, sem_s):
        wid = lax.axis_index("s") * NC + lax.axis_index("c")
        g0 = wid * n_ch
        iota = lax.iota(jnp.int32, L)
        e_pat = [(iota + c) & (L - 1) for c in range(L)]
        d_pat = [((e >> 3) << 12) + ((e & 7) << 7) + iota for e in e_pat]

        def idx_load(chunk, bb):
            return pltpu.async_copy(
                ids_hbm.at[pl.ds((g0 + chunk) * CH, CH)], idx_v[bb],
                sem_i[bb])

        def gather(bb):
            return pltpu.async_copy(
                table_hbm.at[idx_v[bb]], rows_v[bb], sem_g[bb])

        def transpose_chunk(bb):
            rows = rows_v[bb]
            dst = t5[bb]

            @plsc.parallel_loop(0, CH // L, unroll=2)
            def body(tb):
                row_idx = 16 * tb + iota
                btl = tb >> 3
                ttm = (tb & 7) << 4
                base0 = (btl << 10) + ttm
                for dd0 in (0, d // 2):
                    base = base0 + (dd0 >> 3) * blk
                    for c in range(L):
                        col = e_pat[c] + dd0 if dd0 else e_pat[c]
                        v = plsc.load_gather(rows, [row_idx, col])
                        plsc.store_scatter(dst, [d_pat[c] + base], v)

        def stores(chunk, bb):
            g = g0 + chunk
            s_idx = g // ch_per_s
            c = g - s_idx * ch_per_s
            hs = []
            for di in range(dt):
                dpos = s_idx * s_stride + di * blk * ch_per_s + c * blk
                hs.append(pltpu.async_copy(
                    t5[bb].at[pl.ds(di * blk, blk)],
                    out_hbm.at[pl.ds(dpos, blk)], sem_s[bb]))
            return hs

        def wait_stores(bb):
            for di in range(dt):
                pltpu.make_async_copy(
                    t5[bb].at[pl.ds(di * blk, blk)], dummy_hbm,
                    sem_s[bb]).wait()

        for bb in range(2):
            for di in range(dt):
                pltpu.async_copy(
                    t5[bb].at[pl.ds(di * blk, blk)], dummy_hbm, sem_s[bb])
        idx_load(0, 0)
        idx_load(1, 1)
        pltpu.make_async_copy(
            ids_hbm.at[pl.ds(0, CH)], idx_v[0], sem_i[0]).wait()
        gather(0)

        def pair(p, carry):
            a = 2 * p
            pltpu.make_async_copy(
                table_hbm.at[idx_v[0]], rows_v[0], sem_g[0]).wait()
            idx_load(jnp.minimum(a + 2, n_ch - 1), 0)
            pltpu.make_async_copy(
                ids_hbm.at[pl.ds(0, CH)], idx_v[1], sem_i[1]).wait()
            gather(1)
            wait_stores(0)
            transpose_chunk(0)
            stores(a, 0)
            pltpu.make_async_copy(
                table_hbm.at[idx_v[1]], rows_v[1], sem_g[1]).wait()
            idx_load(jnp.minimum(a + 3, n_ch - 1), 1)
            pltpu.make_async_copy(
                ids_hbm.at[pl.ds(0, CH)], idx_v[0], sem_i[0]).wait()
            gather(0)
            wait_stores(1)
            transpose_chunk(1)
            stores(a + 1, 1)
            return carry

        lax.fori_loop(0, n_pair, pair, 0)

        pltpu.make_async_copy(
            table_hbm.at[idx_v[0]], rows_v[0], sem_g[0]).wait()
        pltpu.make_async_copy(
            ids_hbm.at[pl.ds(0, CH)], idx_v[1], sem_i[1]).wait()
        wait_stores(0)
        wait_stores(1)

    return k(ids, table)


def kernel(token_ids, table):
    b, s = token_ids.shape
    d = table.shape[1]
    n = b * s
    ids = token_ids.T.reshape(n).astype(jnp.int32)
    flat, _ = _emb_lookup(ids, table, n, d, b)
    out5 = flat.reshape(s, d // 8, b // 128, 8, 128)
    out3 = out5.transpose(0, 1, 3, 2, 4).reshape(s, d, b)
    return out3.transpose(2, 0, 1)


# speedup vs baseline: 4.6319x; 1.0246x over previous
import functools

import jax
import jax.numpy as jnp
from jax import lax
from jax.experimental import pallas as pl
from jax.experimental.pallas import tpu as pltpu
from jax.experimental.pallas import tpu_sc as plsc

NC = 2
NS = 16
NW = NC * NS
CH = 512
L = 16


@functools.partial(jax.jit, static_argnums=(2, 3, 4))
def _emb_lookup(ids, table, n, d, b_sz):
    n_per_w = n // NW
    n_ch = n_per_w // CH
    n_pair = n_ch // 2
    ch_per_s = b_sz // CH
    dt = d // 8
    blk = CH * d // dt
    s_stride = b_sz * d
    mesh = plsc.VectorSubcoreMesh(
        core_axis_name="c", subcore_axis_name="s",
        num_cores=NC, num_subcores=NS)

    @functools.partial(
        pl.kernel,
        out_type=(
            jax.ShapeDtypeStruct((n * d,), jnp.float32),
            jax.ShapeDtypeStruct((blk,), jnp.float32),
        ),
        mesh=mesh,
        scratch_types=[
            [pltpu.VMEM((CH,), jnp.int32) for _ in range(2)],
            [pltpu.VMEM((CH, d), jnp.float32) for _ in range(2)],
            [pltpu.VMEM((CH * d,), jnp.float32) for _ in range(2)],
            [pltpu.SemaphoreType.DMA for _ in range(2)],
            [pltpu.SemaphoreType.DMA for _ in range(2)],
            [pltpu.SemaphoreType.DMA for _ in range(2)],
        ],
        compiler_params=pltpu.CompilerParams(
            use_tc_tiling_on_sc=False, needs_layout_passes=False),
    )
    def k(ids_hbm, table_hbm, out_hbm, dummy_hbm, idx_v, rows_v, t5, sem_i,
          sem_g, sem_s):
        wid = lax.axis_index("s") * NC + lax.axis_index("c")
        g0 = wid * n_ch
        iota = lax.iota(jnp.int32, L)
        e_pat = [(iota + c) & (L - 1) for c in range(L)]
        d_pat = [((e >> 3) << 12) + ((e & 7) << 7) + iota for e in e_pat]

        def idx_load(chunk, bb):
            return pltpu.async_copy(
                ids_hbm.at[pl.ds((g0 + chunk) * CH, CH)], idx_v[bb],
                sem_i[bb])

        def gather(bb):
            return pltpu.async_copy(
                table_hbm.at[idx_v[bb]], rows_v[bb], sem_g[bb])

        def transpose_chunk(bb):
            rows = rows_v[bb]
            dst = t5[bb]

            @plsc.parallel_loop(0, CH // L, unroll=4)
            def body(tb):
                row_idx = 16 * tb + iota
                btl = tb >> 3
                ttm = (tb & 7) << 4
                base0 = (btl << 10) + ttm
                for dd0 in (0, d // 2):
                    base = base0 + (dd0 >> 3) * blk
                    for c in range(L):
                        col = e_pat[c] + dd0 if dd0 else e_pat[c]
                        v = plsc.load_gather(rows, [row_idx, col])
                        plsc.store_scatter(dst, [d_pat[c] + base], v)

        def stores(chunk, bb):
            g = g0 + chunk
            s_idx = g // ch_per_s
            c = g - s_idx * ch_per_s
            hs = []
            for di in range(dt):
                dpos = s_idx * s_stride + di * blk * ch_per_s + c * blk
                hs.append(pltpu.async_copy(
                    t5[bb].at[pl.ds(di * blk, blk)],
                    out_hbm.at[pl.ds(dpos, blk)], sem_s[bb]))
            return hs

        def wait_stores(bb):
            for di in range(dt):
                pltpu.make_async_copy(
                    t5[bb].at[pl.ds(di * blk, blk)], dummy_hbm,
                    sem_s[bb]).wait()

        for bb in range(2):
            for di in range(dt):
                pltpu.async_copy(
                    t5[bb].at[pl.ds(di * blk, blk)], dummy_hbm, sem_s[bb])
        idx_load(0, 0)
        idx_load(1, 1)
        pltpu.make_async_copy(
            ids_hbm.at[pl.ds(0, CH)], idx_v[0], sem_i[0]).wait()
        gather(0)

        def pair(p, carry):
            a = 2 * p
            pltpu.make_async_copy(
                table_hbm.at[idx_v[0]], rows_v[0], sem_g[0]).wait()
            idx_load(jnp.minimum(a + 2, n_ch - 1), 0)
            pltpu.make_async_copy(
                ids_hbm.at[pl.ds(0, CH)], idx_v[1], sem_i[1]).wait()
            gather(1)
            wait_stores(0)
            transpose_chunk(0)
            stores(a, 0)
            pltpu.make_async_copy(
                table_hbm.at[idx_v[1]], rows_v[1], sem_g[1]).wait()
            idx_load(jnp.minimum(a + 3, n_ch - 1), 1)
            pltpu.make_async_copy(
                ids_hbm.at[pl.ds(0, CH)], idx_v[0], sem_i[0]).wait()
            gather(0)
            wait_stores(1)
            transpose_chunk(1)
            stores(a + 1, 1)
            return carry

        lax.fori_loop(0, n_pair, pair, 0)

        pltpu.make_async_copy(
            table_hbm.at[idx_v[0]], rows_v[0], sem_g[0]).wait()
        pltpu.make_async_copy(
            ids_hbm.at[pl.ds(0, CH)], idx_v[1], sem_i[1]).wait()
        wait_stores(0)
        wait_stores(1)

    return k(ids, table)


def kernel(token_ids, table):
    b, s = token_ids.shape
    d = table.shape[1]
    n = b * s
    ids = token_ids.T.reshape(n).astype(jnp.int32)
    flat, _ = _emb_lookup(ids, table, n, d, b)
    out5 = flat.reshape(s, d // 8, b // 128, 8, 128)
    out3 = out5.transpose(0, 1, 3, 2, 4).reshape(s, d, b)
    return out3.transpose(2, 0, 1)
